# Initial kernel scaffold; baseline (speedup 1.0000x reference)
import jax
import jax.numpy as jnp
from jax.experimental import pallas as pl


def kernel(x, edge_index, W1, b1, W2, b2):
    raise NotImplementedError("write your pallas kernel here")


# speedup vs baseline: 16.5096x; 16.5096x over previous
import functools

import jax
import jax.numpy as jnp
from jax import lax
from jax.experimental import pallas as pl
from jax.experimental.pallas import tpu as pltpu
from jax.experimental.pallas import tpu_sc as plsc

N = 10000
E = 320000
D = 128

NC = 2
TPS = 16
NW = NC * TPS
EPT = E // NW
B = 80
NB = EPT // B
RPT = 624
NTAIL = N - TPS * RPT

_mesh = plsc.VectorSubcoreMesh(core_axis_name="c", subcore_axis_name="s")


@functools.partial(
    pl.kernel,
    out_type=jax.ShapeDtypeStruct((NW, N), jnp.float32),
    mesh=_mesh,
    scratch_types=[
        pltpu.VMEM((EPT,), jnp.int32),
        pltpu.VMEM((N,), jnp.float32),
    ],
    compiler_params=pltpu.CompilerParams(needs_layout_passes=False),
)
def _sc_hist(dst_hbm, out_hbm, dst_v, hist_v):
    c = lax.axis_index("c")
    s = lax.axis_index("s")
    wid = c * TPS + s
    pltpu.sync_copy(dst_hbm.at[pl.ds(wid * EPT, EPT)], dst_v)

    zeros = jnp.zeros((16,), jnp.float32)

    def zbody(i, _):
        hist_v[pl.ds(i * 16, 16)] = zeros
        return 0

    lax.fori_loop(0, N // 16, zbody, 0)

    ones = jnp.ones((16,), jnp.float32)

    def body(i, _):
        idx = dst_v[pl.ds(i * 16, 16)]
        plsc.addupdate_scatter(hist_v, [idx], ones)
        return 0

    lax.fori_loop(0, EPT // 16, body, 0)
    pltpu.sync_copy(hist_v, out_hbm.at[wid])


@functools.partial(
    pl.kernel,
    out_type=jax.ShapeDtypeStruct((NC, N, D), jnp.float32),
    mesh=_mesh,
    scratch_types=[
        pltpu.VMEM((EPT,), jnp.int32),
        pltpu.VMEM((B,), jnp.int32),
        pltpu.VMEM((B, D), jnp.float32),
        pltpu.VMEM_SHARED((N, D), jnp.float32),
        pltpu.SemaphoreType.DMA,
    ],
    compiler_params=pltpu.CompilerParams(needs_layout_passes=False),
)
def _sc_scatter(ht_hbm, src_hbm, dst_hbm, zero_hbm, out_hbm,
                src_v, dst_blk, rows_v, acc_sh, gsem):
    c = lax.axis_index("c")
    s = lax.axis_index("s")
    wid = c * TPS + s
    base = wid * EPT
    pltpu.sync_copy(src_hbm.at[pl.ds(base, EPT)], src_v)
    r0 = s * RPT
    pltpu.sync_copy(zero_hbm.at[pl.ds(r0, RPT)], acc_sh.at[pl.ds(r0, RPT)])

    @pl.when(s == TPS - 1)
    def _():
        pltpu.sync_copy(zero_hbm.at[pl.ds(TPS * RPT, NTAIL)],
                        acc_sh.at[pl.ds(TPS * RPT, NTAIL)])

    plsc.subcore_barrier()

    def body(j, _):
        pltpu.sync_copy(dst_hbm.at[pl.ds(base + j * B, B)], dst_blk)
        pltpu.async_copy(ht_hbm.at[src_v.at[pl.ds(j * B, B)]], rows_v, gsem).wait()
        pltpu.sync_copy(rows_v, acc_sh.at[dst_blk], add=True)
        return 0

    lax.fori_loop(0, NB, body, 0)
    plsc.subcore_barrier()
    pltpu.sync_copy(acc_sh.at[pl.ds(r0, RPT)], out_hbm.at[c, pl.ds(r0, RPT)])

    @pl.when(s == TPS - 1)
    def _():
        pltpu.sync_copy(acc_sh.at[pl.ds(TPS * RPT, NTAIL)],
                        out_hbm.at[c, pl.ds(TPS * RPT, NTAIL)])


RB = 1000


def _tc_dinv_body(hist_ref, dinv_ref):
    deg = jnp.sum(hist_ref[...], axis=0) + 1.0
    dinv_ref[...] = lax.rsqrt(deg).reshape(N, 1)


def _tc_a_body(x_ref, dinv_ref, w_ref, h_ref, ht_ref):
    dinv = dinv_ref[...]
    h = jnp.dot(x_ref[...], w_ref[...], preferred_element_type=jnp.float32)
    h_ref[...] = h
    ht_ref[...] = h * dinv


def _tc_b_body(p_ref, dinv_ref, h1_ref, b1_ref, w2_ref, h2_ref, ht2_ref):
    dinv = dinv_ref[...]
    accum = p_ref[0] + p_ref[1]
    z1 = accum * dinv + h1_ref[...] * (dinv * dinv) + b1_ref[...]
    hrelu = jnp.maximum(z1, 0.0)
    h2 = jnp.dot(hrelu, w2_ref[...], preferred_element_type=jnp.float32)
    h2_ref[...] = h2
    ht2_ref[...] = h2 * dinv


def _tc_c_body(p_ref, dinv_ref, h2_ref, b2_ref, z_ref, ls_ref):
    dinv = dinv_ref[...]
    accum = p_ref[0] + p_ref[1]
    z = accum * dinv + h2_ref[...] * (dinv * dinv) + b2_ref[...]
    z_ref[...] = z
    m = jnp.max(z, axis=-1, keepdims=True)
    lse = jnp.log(jnp.sum(jnp.exp(z - m), axis=-1, keepdims=True)) + m
    ls_ref[...] = z - lse


_row_spec = pl.BlockSpec((RB, D), lambda i: (i, 0))
_dinv_spec = pl.BlockSpec((RB, 1), lambda i: (i, 0))
_w_spec = pl.BlockSpec((D, D), lambda i: (0, 0))
_b_spec = pl.BlockSpec((1, D), lambda i: (0, 0))
_part_spec = pl.BlockSpec((NC, RB, D), lambda i: (0, i, 0))
_grid = (N // RB,)

_tc_dinv = pl.pallas_call(
    _tc_dinv_body,
    out_shape=jax.ShapeDtypeStruct((N, 1), jnp.float32),
)

_tc_a = pl.pallas_call(
    _tc_a_body,
    grid=_grid,
    in_specs=[_row_spec, _dinv_spec, _w_spec],
    out_specs=[_row_spec, _row_spec],
    out_shape=[jax.ShapeDtypeStruct((N, D), jnp.float32)] * 2,
)

_tc_b = pl.pallas_call(
    _tc_b_body,
    grid=_grid,
    in_specs=[_part_spec, _dinv_spec, _row_spec, _b_spec, _w_spec],
    out_specs=[_row_spec, _row_spec],
    out_shape=[jax.ShapeDtypeStruct((N, D), jnp.float32)] * 2,
)

_tc_c = pl.pallas_call(
    _tc_c_body,
    grid=_grid,
    in_specs=[_part_spec, _dinv_spec, _row_spec, _b_spec],
    out_specs=[_row_spec, _row_spec],
    out_shape=[jax.ShapeDtypeStruct((N, D), jnp.float32)] * 2,
)


@jax.jit
def kernel(x, edge_index, W1, b1, W2, b2):
    src = edge_index[0]
    dst = edge_index[1]
    zero_rows = jnp.zeros((N, D), jnp.float32)
    b1r = b1.reshape(1, D)
    b2r = b2.reshape(1, D)

    hist = _sc_hist(dst)
    dinv = _tc_dinv(hist)
    h1, ht1 = _tc_a(x, dinv, W1)
    part1 = _sc_scatter(ht1, src, dst, zero_rows)
    h2, ht2 = _tc_b(part1, dinv, h1, b1r, W2)
    part2 = _sc_scatter(ht2, src, dst, zero_rows)
    z, ls = _tc_c(part2, dinv, h2, b2r)
    return (z, ls)
